# baseline (speedup 1.0000x reference)
import functools

import jax
import jax.numpy as jnp
from jax import lax
from jax.experimental import pallas as pl
from jax.experimental.pallas import tpu as pltpu
from jax.experimental.pallas import tpu_sc as plsc

VOCAB = 100000
DIM = 128
B = 1024
L = 200
N = B * L

_info = plsc.get_sparse_core_info()
NC, NS = _info.num_cores, _info.num_subcores
NW = NC * NS
PER_W = N // NW
CHUNK = 80
NCHUNK = PER_W // CHUNK


NBUF = 8
AHEAD = 2

assert NCHUNK % NBUF == 0 and 1 <= AHEAD < NBUF


def _make_kernel():
    mesh = plsc.VectorSubcoreMesh(core_axis_name="c", subcore_axis_name="s")

    @functools.partial(
        pl.kernel,
        mesh=mesh,
        out_type=jax.ShapeDtypeStruct((N, DIM), jnp.float32),
        scratch_types=[
            pltpu.VMEM((PER_W,), jnp.int32),
        ] + [pltpu.VMEM((CHUNK, DIM), jnp.float32)] * NBUF
          + [pltpu.SemaphoreType.DMA] * (2 * NBUF),
    )
    def gather_kernel(table_hbm, idx_hbm, out_hbm, idx_v, *bufs):
        rows = bufs[:NBUF]
        gs = bufs[NBUF:2 * NBUF]
        ws = bufs[2 * NBUF:]
        wid = lax.axis_index("s") * NC + lax.axis_index("c")
        base = wid * PER_W
        pltpu.sync_copy(idx_hbm.at[pl.ds(base, PER_W)], idx_v)

        def g_desc(j, b):
            return pltpu.make_async_copy(
                table_hbm.at[idx_v.at[pl.ds(j * CHUNK, CHUNK)]], rows[b], gs[b])

        def w_desc(j, b):
            return pltpu.make_async_copy(
                rows[b], out_hbm.at[pl.ds(base + j * CHUNK, CHUNK)], ws[b])

        for j in range(AHEAD):
            g_desc(j, j).start()
        for j in range(NBUF - AHEAD):
            g_desc(j, j % NBUF).wait()
            w_desc(j, j % NBUF).start()
            g_desc(j + AHEAD, (j + AHEAD) % NBUF).start()

        def body(i, carry):
            for o in range(NBUF):
                j = (NBUF - AHEAD) + i * NBUF + o
                b = (NBUF - AHEAD + o) % NBUF
                g_desc(j, b).wait()
                w_desc(j, b).start()
                ba = (b + AHEAD) % NBUF
                w_desc(j + AHEAD - NBUF, ba).wait()
                g_desc(j + AHEAD, ba).start()
            return carry

        lax.fori_loop(0, (NCHUNK - NBUF) // NBUF, body, 0)

        for j in range(NCHUNK - AHEAD, NCHUNK):
            b = j % NBUF
            g_desc(j, b).wait()
            w_desc(j, b).start()
            w_desc(j + AHEAD - NBUF, (b + AHEAD) % NBUF).wait()
        for j in range(NCHUNK - NBUF + AHEAD, NCHUNK):
            w_desc(j, j % NBUF).wait()

    return gather_kernel


_gather = _make_kernel()


def kernel(idx, W):
    flat = idx.reshape(N).astype(jnp.int32)
    out = _gather(W, flat)
    return out.reshape(B, L, DIM)


# speedup vs baseline: 1.0432x; 1.0432x over previous
import functools

import jax
import jax.numpy as jnp
from jax import lax
from jax.experimental import pallas as pl
from jax.experimental.pallas import tpu as pltpu
from jax.experimental.pallas import tpu_sc as plsc

VOCAB = 100000
DIM = 128
B = 1024
L = 200
N = B * L

_info = plsc.get_sparse_core_info()
NC, NS = _info.num_cores, _info.num_subcores
NW = NC * NS
PER_W = N // NW
CHUNK = 80
NCHUNK = PER_W // CHUNK


NBUF = 8
AHEAD = 6

assert NCHUNK % NBUF == 0 and 1 <= AHEAD < NBUF


def _make_kernel():
    mesh = plsc.VectorSubcoreMesh(core_axis_name="c", subcore_axis_name="s")

    @functools.partial(
        pl.kernel,
        mesh=mesh,
        out_type=jax.ShapeDtypeStruct((N, DIM), jnp.float32),
        scratch_types=[
            pltpu.VMEM((PER_W,), jnp.int32),
        ] + [pltpu.VMEM((CHUNK, DIM), jnp.float32)] * NBUF
          + [pltpu.SemaphoreType.DMA] * (2 * NBUF),
    )
    def gather_kernel(table_hbm, idx_hbm, out_hbm, idx_v, *bufs):
        rows = bufs[:NBUF]
        gs = bufs[NBUF:2 * NBUF]
        ws = bufs[2 * NBUF:]
        wid = lax.axis_index("s") * NC + lax.axis_index("c")
        base = wid * PER_W
        pltpu.sync_copy(idx_hbm.at[pl.ds(base, PER_W)], idx_v)

        def g_desc(j, b):
            return pltpu.make_async_copy(
                table_hbm.at[idx_v.at[pl.ds(j * CHUNK, CHUNK)]], rows[b], gs[b])

        def w_desc(j, b):
            return pltpu.make_async_copy(
                rows[b], out_hbm.at[pl.ds(base + j * CHUNK, CHUNK)], ws[b])

        for j in range(AHEAD):
            g_desc(j, j).start()
        for j in range(NBUF - AHEAD):
            g_desc(j, j % NBUF).wait()
            w_desc(j, j % NBUF).start()
            g_desc(j + AHEAD, (j + AHEAD) % NBUF).start()

        def body(i, carry):
            for o in range(NBUF):
                j = (NBUF - AHEAD) + i * NBUF + o
                b = (NBUF - AHEAD + o) % NBUF
                g_desc(j, b).wait()
                w_desc(j, b).start()
                ba = (b + AHEAD) % NBUF
                w_desc(j + AHEAD - NBUF, ba).wait()
                g_desc(j + AHEAD, ba).start()
            return carry

        lax.fori_loop(0, (NCHUNK - NBUF) // NBUF, body, 0)

        for j in range(NCHUNK - AHEAD, NCHUNK):
            b = j % NBUF
            g_desc(j, b).wait()
            w_desc(j, b).start()
            w_desc(j + AHEAD - NBUF, (b + AHEAD) % NBUF).wait()
        for j in range(NCHUNK - NBUF + AHEAD, NCHUNK):
            w_desc(j, j % NBUF).wait()

    return gather_kernel


_gather = _make_kernel()


def kernel(idx, W):
    flat = idx.reshape(N).astype(jnp.int32)
    out = _gather(W, flat)
    return out.reshape(B, L, DIM)


# speedup vs baseline: 1.0505x; 1.0071x over previous
import functools

import jax
import jax.numpy as jnp
from jax import lax
from jax.experimental import pallas as pl
from jax.experimental.pallas import tpu as pltpu
from jax.experimental.pallas import tpu_sc as plsc

VOCAB = 100000
DIM = 128
B = 1024
L = 200
N = B * L

_info = plsc.get_sparse_core_info()
NC, NS = _info.num_cores, _info.num_subcores
NW = NC * NS
PER_W = N // NW
CHUNK = 80
NCHUNK = PER_W // CHUNK


NBUF = 10
AHEAD = 8

assert NCHUNK % NBUF == 0 and 1 <= AHEAD < NBUF


def _make_kernel():
    mesh = plsc.VectorSubcoreMesh(core_axis_name="c", subcore_axis_name="s")

    @functools.partial(
        pl.kernel,
        mesh=mesh,
        out_type=jax.ShapeDtypeStruct((N, DIM), jnp.float32),
        scratch_types=[
            pltpu.VMEM((PER_W,), jnp.int32),
        ] + [pltpu.VMEM((CHUNK, DIM), jnp.float32)] * NBUF
          + [pltpu.SemaphoreType.DMA] * (2 * NBUF),
    )
    def gather_kernel(table_hbm, idx_hbm, out_hbm, idx_v, *bufs):
        rows = bufs[:NBUF]
        gs = bufs[NBUF:2 * NBUF]
        ws = bufs[2 * NBUF:]
        wid = lax.axis_index("s") * NC + lax.axis_index("c")
        base = wid * PER_W
        pltpu.sync_copy(idx_hbm.at[pl.ds(base, PER_W)], idx_v)

        def g_desc(j, b):
            return pltpu.make_async_copy(
                table_hbm.at[idx_v.at[pl.ds(j * CHUNK, CHUNK)]], rows[b], gs[b])

        def w_desc(j, b):
            return pltpu.make_async_copy(
                rows[b], out_hbm.at[pl.ds(base + j * CHUNK, CHUNK)], ws[b])

        for j in range(AHEAD):
            g_desc(j, j).start()
        for j in range(NBUF - AHEAD):
            g_desc(j, j % NBUF).wait()
            w_desc(j, j % NBUF).start()
            g_desc(j + AHEAD, (j + AHEAD) % NBUF).start()

        def body(i, carry):
            for o in range(NBUF):
                j = (NBUF - AHEAD) + i * NBUF + o
                b = (NBUF - AHEAD + o) % NBUF
                g_desc(j, b).wait()
                w_desc(j, b).start()
                ba = (b + AHEAD) % NBUF
                w_desc(j + AHEAD - NBUF, ba).wait()
                g_desc(j + AHEAD, ba).start()
            return carry

        lax.fori_loop(0, (NCHUNK - NBUF) // NBUF, body, 0)

        for j in range(NCHUNK - AHEAD, NCHUNK):
            b = j % NBUF
            g_desc(j, b).wait()
            w_desc(j, b).start()
            w_desc(j + AHEAD - NBUF, (b + AHEAD) % NBUF).wait()
        for j in range(NCHUNK - NBUF + AHEAD, NCHUNK):
            w_desc(j, j % NBUF).wait()

    return gather_kernel


_gather = _make_kernel()


def kernel(idx, W):
    flat = idx.reshape(N).astype(jnp.int32)
    out = _gather(W, flat)
    return out.reshape(B, L, DIM)
